# baseline (speedup 1.0000x reference)
import functools

import jax
import jax.numpy as jnp
from jax import lax
from jax.experimental import pallas as pl
from jax.experimental.pallas import tpu as pltpu
from jax.experimental.pallas import tpu_sc as plsc

DIN = 256
D = 256
N = 4096
CH = 8
NW = 32
NPW = N // NW
NODES_PER_CHUNK = 8
CHUNKS = NPW // NODES_PER_CHUNK
ROWS_PER_CHUNK = NODES_PER_CHUNK * CH
TAB_ROWS = N + 512
SEG = 16
NSEG = D // SEG


def _wx_body(x_ref, w_ref, b_ref, o_ref):
    o_ref[...] = (
        jnp.dot(x_ref[...], w_ref[...], preferred_element_type=jnp.float32)
        + b_ref[...]
    )


def _wx_matmul(x2, W_kernel, W_bias):
    M = x2.shape[0]
    BM = 1024
    return pl.pallas_call(
        _wx_body,
        grid=(M // BM,),
        in_specs=[
            pl.BlockSpec((BM, DIN), lambda i: (i, 0)),
            pl.BlockSpec((DIN, 4 * D), lambda i: (0, 0)),
            pl.BlockSpec((1, 4 * D), lambda i: (0, 0)),
        ],
        out_specs=pl.BlockSpec((BM, 4 * D), lambda i: (i, 0)),
        out_shape=jax.ShapeDtypeStruct((M, 4 * D), jnp.float32),
    )(x2, W_kernel, W_bias.reshape(1, 4 * D))


def _level_body(wx_ref, scfo_ref, uiuo_ref, uf_ref, h_ref, c_ref, tab_ref):
    i = pl.program_id(0)
    d = D
    h_sum = scfo_ref[:, :d]
    fco = scfo_ref[:, d:]
    iuo = jnp.dot(h_sum, uiuo_ref[...], preferred_element_type=jnp.float32)
    wx = wx_ref[...]
    gi = jax.nn.sigmoid(iuo[:, :d] + wx[:, d:2 * d])
    gu = jnp.tanh(iuo[:, d:2 * d] + wx[:, 2 * d:3 * d])
    go = jax.nn.sigmoid(iuo[:, 2 * d:] + wx[:, 3 * d:])
    new_c = gi * gu + fco
    new_h = go * jnp.tanh(new_c)
    hu = jnp.dot(new_h, uf_ref[...], preferred_element_type=jnp.float32)
    h_ref[...] = new_h
    c_ref[...] = new_c
    live = (i < 8).astype(jnp.float32)
    tab_ref[:, :d] = new_h * live
    tab_ref[:, d:2 * d] = new_c * live
    tab_ref[:, 2 * d:] = hu * live


def _tc_level(wx_t, scfo, uiuo, uf):
    BN = 512
    return pl.pallas_call(
        _level_body,
        grid=(TAB_ROWS // BN,),
        in_specs=[
            pl.BlockSpec((BN, 4 * D), lambda i: (jnp.minimum(i, 7), 0)),
            pl.BlockSpec((BN, 2 * D), lambda i: (jnp.minimum(i, 7), 0)),
            pl.BlockSpec((D, 3 * D), lambda i: (0, 0)),
            pl.BlockSpec((D, D), lambda i: (0, 0)),
        ],
        out_specs=[
            pl.BlockSpec((BN, D), lambda i: (jnp.minimum(i, 7), 0)),
            pl.BlockSpec((BN, D), lambda i: (jnp.minimum(i, 7), 0)),
            pl.BlockSpec((BN, 3 * D), lambda i: (i, 0)),
        ],
        out_shape=[
            jax.ShapeDtypeStruct((N, D), jnp.float32),
            jax.ShapeDtypeStruct((N, D), jnp.float32),
            jax.ShapeDtypeStruct((TAB_ROWS, 3 * D), jnp.float32),
        ],
    )(wx_t, scfo, uiuo, uf)


def _sc_body(tab_hbm, safe_hbm, wf_hbm, out_hbm, idx_v, rows_v, wf_v, acc_v,
             sem):
    wid = lax.axis_index("s") * 2 + lax.axis_index("c")
    nbase = wid * NPW
    pltpu.sync_copy(safe_hbm.at[wid], idx_v)

    def chunk_body(c, carry):
        pltpu.async_copy(tab_hbm.at[idx_v.at[c]], rows_v, sem).wait()
        pltpu.sync_copy(
            wf_hbm.at[pl.ds(nbase + c * NODES_PER_CHUNK, NODES_PER_CHUNK)],
            wf_v)

        def node_body(n, carry2):
            wfs = [wf_v[n, pl.ds(j * SEG, SEG)] for j in range(NSEG)]

            def child_body(k, accs):
                accs_h, accs_f = accs
                r = n * CH + k
                new_h = []
                new_f = []
                for j in range(NSEG):
                    off = j * SEG
                    hseg = rows_v[r, pl.ds(off, SEG)]
                    cseg = rows_v[r, pl.ds(D + off, SEG)]
                    useg = rows_v[r, pl.ds(2 * D + off, SEG)]
                    s = 1.0 / (1.0 + jnp.exp(-(wfs[j] + useg)))
                    new_h.append(accs_h[j] + hseg)
                    new_f.append(accs_f[j] + s * cseg)
                return (new_h, new_f)

            zero = jnp.zeros((SEG,), jnp.float32)
            init = ([zero] * NSEG, [zero] * NSEG)
            accs_h, accs_f = lax.fori_loop(0, CH, child_body, init)
            for j in range(NSEG):
                acc_v[n, pl.ds(j * SEG, SEG)] = accs_h[j]
                acc_v[n, pl.ds(D + j * SEG, SEG)] = accs_f[j]
            return carry2

        lax.fori_loop(0, NODES_PER_CHUNK, node_body, 0)
        pltpu.sync_copy(
            acc_v,
            out_hbm.at[pl.ds(nbase + c * NODES_PER_CHUNK, NODES_PER_CHUNK)])
        return carry

    lax.fori_loop(0, CHUNKS, chunk_body, 0)


_sc_gather = pl.kernel(
    _sc_body,
    out_type=jax.ShapeDtypeStruct((N, 2 * D), jnp.float32),
    mesh=plsc.VectorSubcoreMesh(core_axis_name="c", subcore_axis_name="s"),
    scratch_types=[
        pltpu.VMEM((CHUNKS, ROWS_PER_CHUNK), jnp.int32),
        pltpu.VMEM((ROWS_PER_CHUNK, 3 * D), jnp.float32),
        pltpu.VMEM((NODES_PER_CHUNK, D), jnp.float32),
        pltpu.VMEM((NODES_PER_CHUNK, 2 * D), jnp.float32),
        pltpu.SemaphoreType.DMA,
    ],
)


def kernel(tensor, indices, W_kernel, W_bias, Uf_kernel, Uiuo_kernel):
    L = tensor.shape[0]
    d = D
    Wx = _wx_matmul(tensor.reshape(L * N, DIN), W_kernel, W_bias)
    Wx = Wx.reshape(L, N, 4 * d)
    safe = jnp.where(indices >= 1, indices - 1, N).astype(jnp.int32)
    safe = safe.reshape(L, NW, CHUNKS, ROWS_PER_CHUNK)

    res_h, res_c = [], []
    tab = None
    for t in range(L):
        if t == 0:
            scfo = jnp.zeros((N, 2 * d), jnp.float32)
        else:
            scfo = _sc_gather(tab, safe[t], Wx[t, :, :d])
        h_t, c_t, tab = _tc_level(Wx[t], scfo, Uiuo_kernel, Uf_kernel)
        res_h.append(h_t)
        res_c.append(c_t)
    return (jnp.stack(res_h), jnp.stack(res_c))


# speedup vs baseline: 1.0283x; 1.0283x over previous
import functools

import jax
import jax.numpy as jnp
from jax import lax
from jax.experimental import pallas as pl
from jax.experimental.pallas import tpu as pltpu
from jax.experimental.pallas import tpu_sc as plsc

DIN = 256
D = 256
N = 4096
CH = 8
NW = 32
NPW = N // NW
NODES_PER_CHUNK = 8
CHUNKS = NPW // NODES_PER_CHUNK
ROWS_PER_CHUNK = NODES_PER_CHUNK * CH
TAB_ROWS = N + 512
SEG = 16
NSEG = D // SEG


def _wx_body(x_ref, w_ref, b_ref, o_ref):
    o_ref[...] = (
        jnp.dot(x_ref[...], w_ref[...], preferred_element_type=jnp.float32)
        + b_ref[...]
    )


def _wx_matmul(x2, W_kernel, W_bias):
    M = x2.shape[0]
    BM = 1024
    return pl.pallas_call(
        _wx_body,
        grid=(M // BM,),
        in_specs=[
            pl.BlockSpec((BM, DIN), lambda i: (i, 0)),
            pl.BlockSpec((DIN, 4 * D), lambda i: (0, 0)),
            pl.BlockSpec((1, 4 * D), lambda i: (0, 0)),
        ],
        out_specs=pl.BlockSpec((BM, 4 * D), lambda i: (i, 0)),
        out_shape=jax.ShapeDtypeStruct((M, 4 * D), jnp.float32),
    )(x2, W_kernel, W_bias.reshape(1, 4 * D))


def _level_body(wx_ref, scfo_ref, uiuo_ref, uf_ref, h_ref, c_ref, tab_ref):
    i = pl.program_id(0)
    d = D
    h_sum = scfo_ref[:, :d]
    fco = scfo_ref[:, d:]
    iuo = jnp.dot(h_sum, uiuo_ref[...], preferred_element_type=jnp.float32)
    wx = wx_ref[...]
    gi = jax.nn.sigmoid(iuo[:, :d] + wx[:, d:2 * d])
    gu = jnp.tanh(iuo[:, d:2 * d] + wx[:, 2 * d:3 * d])
    go = jax.nn.sigmoid(iuo[:, 2 * d:] + wx[:, 3 * d:])
    new_c = gi * gu + fco
    new_h = go * jnp.tanh(new_c)
    hu = jnp.dot(new_h, uf_ref[...], preferred_element_type=jnp.float32)
    h_ref[...] = new_h
    c_ref[...] = new_c
    live = (i < 8).astype(jnp.float32)
    tab_ref[:, :d] = new_h * live
    tab_ref[:, d:2 * d] = new_c * live
    tab_ref[:, 2 * d:] = hu * live


def _tc_level(wx_t, scfo, uiuo, uf):
    BN = 512
    return pl.pallas_call(
        _level_body,
        grid=(TAB_ROWS // BN,),
        in_specs=[
            pl.BlockSpec((BN, 4 * D), lambda i: (jnp.minimum(i, 7), 0)),
            pl.BlockSpec((BN, 2 * D), lambda i: (jnp.minimum(i, 7), 0)),
            pl.BlockSpec((D, 3 * D), lambda i: (0, 0)),
            pl.BlockSpec((D, D), lambda i: (0, 0)),
        ],
        out_specs=[
            pl.BlockSpec((BN, D), lambda i: (jnp.minimum(i, 7), 0)),
            pl.BlockSpec((BN, D), lambda i: (jnp.minimum(i, 7), 0)),
            pl.BlockSpec((BN, 3 * D), lambda i: (i, 0)),
        ],
        out_shape=[
            jax.ShapeDtypeStruct((N, D), jnp.float32),
            jax.ShapeDtypeStruct((N, D), jnp.float32),
            jax.ShapeDtypeStruct((TAB_ROWS, 3 * D), jnp.float32),
        ],
    )(wx_t, scfo, uiuo, uf)


def _sc_body(tab_hbm, safe_hbm, wf_hbm, out_hbm, idx_v, rows_v, wf_v, acc_v,
             sem_g0, sem_g1, sem_o0, sem_o1):
    wid = lax.axis_index("s") * 2 + lax.axis_index("c")
    nbase = wid * NPW
    sem_g = (sem_g0, sem_g1)
    sem_o = (sem_o0, sem_o1)
    pltpu.sync_copy(safe_hbm.at[wid], idx_v)

    def start_gather(ch, b):
        pltpu.async_copy(tab_hbm.at[idx_v.at[ch]], rows_v.at[b], sem_g[b])
        pltpu.async_copy(
            wf_hbm.at[pl.ds(nbase + ch * NODES_PER_CHUNK, NODES_PER_CHUNK)],
            wf_v.at[b], sem_g[b])

    start_gather(0, 0)

    def compute_chunk(ch, b):
        nxt = ch + 1

        @pl.when(nxt < CHUNKS)
        def _():
            start_gather(nxt, b ^ 1)

        pltpu.make_async_copy(tab_hbm.at[idx_v.at[ch]], rows_v.at[b],
                              sem_g[b]).wait()
        pltpu.make_async_copy(
            wf_hbm.at[pl.ds(nbase + ch * NODES_PER_CHUNK, NODES_PER_CHUNK)],
            wf_v.at[b], sem_g[b]).wait()

        @pl.when(ch >= 2)
        def _():
            pltpu.make_async_copy(
                acc_v.at[b],
                out_hbm.at[pl.ds(nbase + (ch - 2) * NODES_PER_CHUNK,
                                 NODES_PER_CHUNK)],
                sem_o[b]).wait()

        def node_body(n, carry):
            r0 = n * CH
            for jg in range(0, NSEG, 4):
                G = list(range(jg, jg + 4))
                wf4 = [wf_v[b, n, pl.ds(j * SEG, SEG)] for j in G]
                acch = [rows_v[b, r0, pl.ds(j * SEG, SEG)] for j in G]
                u4 = [rows_v[b, r0, pl.ds(2 * D + j * SEG, SEG)] for j in G]
                c4 = [rows_v[b, r0, pl.ds(D + j * SEG, SEG)] for j in G]
                e4 = [jnp.exp(wf4[i] + u4[i]) for i in range(4)]
                accf = [c4[i] / (1.0 + e4[i]) for i in range(4)]
                for k in range(1, CH):
                    r = r0 + k
                    h2 = [rows_v[b, r, pl.ds(j * SEG, SEG)] for j in G]
                    u2 = [rows_v[b, r, pl.ds(2 * D + j * SEG, SEG)]
                          for j in G]
                    c2 = [rows_v[b, r, pl.ds(D + j * SEG, SEG)] for j in G]
                    acch = [acch[i] + h2[i] for i in range(4)]
                    e2 = [jnp.exp(wf4[i] + u2[i]) for i in range(4)]
                    s2 = [c2[i] / (1.0 + e2[i]) for i in range(4)]
                    accf = [accf[i] + s2[i] for i in range(4)]
                for i, j in enumerate(G):
                    acc_v[b, n, pl.ds(j * SEG, SEG)] = acch[i]
                    acc_v[b, n, pl.ds(D + j * SEG, SEG)] = accf[i]
            return carry

        lax.fori_loop(0, NODES_PER_CHUNK, node_body, 0)
        pltpu.async_copy(
            acc_v.at[b],
            out_hbm.at[pl.ds(nbase + ch * NODES_PER_CHUNK, NODES_PER_CHUNK)],
            sem_o[b])

    def pair_body(c2, carry):
        for b in range(2):
            compute_chunk(c2 * 2 + b, b)
        return carry

    lax.fori_loop(0, CHUNKS // 2, pair_body, 0)
    for b in range(2):
        pltpu.make_async_copy(
            acc_v.at[b],
            out_hbm.at[pl.ds(nbase + (CHUNKS - 2 + b) * NODES_PER_CHUNK,
                             NODES_PER_CHUNK)],
            sem_o[b]).wait()


_sc_gather = pl.kernel(
    _sc_body,
    out_type=jax.ShapeDtypeStruct((N, 2 * D), jnp.float32),
    mesh=plsc.VectorSubcoreMesh(core_axis_name="c", subcore_axis_name="s"),
    scratch_types=[
        pltpu.VMEM((CHUNKS, ROWS_PER_CHUNK), jnp.int32),
        pltpu.VMEM((2, ROWS_PER_CHUNK, 3 * D), jnp.float32),
        pltpu.VMEM((2, NODES_PER_CHUNK, D), jnp.float32),
        pltpu.VMEM((2, NODES_PER_CHUNK, 2 * D), jnp.float32),
        pltpu.SemaphoreType.DMA,
        pltpu.SemaphoreType.DMA,
        pltpu.SemaphoreType.DMA,
        pltpu.SemaphoreType.DMA,
    ],
)


def kernel(tensor, indices, W_kernel, W_bias, Uf_kernel, Uiuo_kernel):
    L = tensor.shape[0]
    d = D
    W_kernel = jnp.concatenate([-W_kernel[:, :d], W_kernel[:, d:]], axis=1)
    W_bias = jnp.concatenate([-W_bias[:d], W_bias[d:]])
    Uf_scaled = -Uf_kernel
    Wx = _wx_matmul(tensor.reshape(L * N, DIN), W_kernel, W_bias)
    Wx = Wx.reshape(L, N, 4 * d)
    safe = jnp.where(indices >= 1, indices - 1, N).astype(jnp.int32)
    safe = safe.reshape(L, NW, CHUNKS, ROWS_PER_CHUNK)

    res_h, res_c = [], []
    tab = None
    for t in range(L):
        if t == 0:
            scfo = jnp.zeros((N, 2 * d), jnp.float32)
        else:
            scfo = _sc_gather(tab, safe[t], Wx[t, :, :d])
        h_t, c_t, tab = _tc_level(Wx[t], scfo, Uiuo_kernel, Uf_scaled)
        res_h.append(h_t)
        res_c.append(c_t)
    return (jnp.stack(res_h), jnp.stack(res_c))
